# baseline (speedup 1.0000x reference)
import functools

import numpy as np
import jax
import jax.numpy as jnp
from jax import lax
from jax.experimental import pallas as pl
from jax.experimental.pallas import tpu as pltpu
from jax.experimental.pallas import tpu_sc as plsc

_EPS = np.float32(1e-06)
_PAD = np.float32(-1.0)
_NEG_INF = np.float32(-np.inf)
_LN2 = np.float32(0.6931471805599453)
_C3 = np.float32(1.0 / 3.0)
_C5 = np.float32(1.0 / 5.0)
_C7 = np.float32(1.0 / 7.0)
_C9 = np.float32(1.0 / 9.0)

_NC, _NS, _LANES = 2, 16, 16
_NW = _NC * _NS
_NV = 16

_PAD_KEY = np.float32(np.inf)
_PADBITS = np.int32(np.int64(0xBF800000) - (1 << 32))
_INFBITS = np.int32(0x7F800000)
_HI24 = np.int32(np.int64(0xFFFFFF00) - (1 << 32))
_LO8 = np.int32(0xFF)


def _swlog(x):
    bits = plsc.bitcast(x, jnp.int32)
    e = ((bits >> 23) & 0xFF) - 127
    mant = plsc.bitcast((bits & 0x007FFFFF) | 0x3F800000, jnp.float32)
    z = (mant - np.float32(1.0)) / (mant + np.float32(1.0))
    w = z * z
    poly = (np.float32(2.0) * z *
            (np.float32(1.0) + w * (_C3 + w * (_C5 + w * (_C7 + w * _C9)))))
    return e.astype(jnp.float32) * _LN2 + poly


_TBASE = int(np.float32(1e-6).view(np.int32)) >> 15
_TLEN = 7168


def _log_tables():
    idx = np.arange(_TLEN, dtype=np.int64)
    bits0 = (idx + _TBASE) << 15
    x0 = bits0.astype(np.int64).astype(np.int32).view(np.float32).astype(np.float64)
    x1 = (bits0 + 32768).astype(np.int64).astype(np.int32).view(np.float32).astype(np.float64)
    t = np.log(x0)
    d = (np.log(x1) - t) / 32768.0
    return t.astype(np.float32), d.astype(np.float32)


class _Vr:
    __slots__ = ("k", "ispad", "dirty")

    def __init__(self, k, ispad=False, dirty=True):
        self.k, self.ispad, self.dirty = k, ispad, dirty


def _vsort(x):
    if x.ispad or not x.dirty:
        return x
    return _Vr(jnp.sort(x.k), dirty=False)


def _vrev(x):
    if x.ispad:
        return x
    return _Vr(lax.rev(x.k, (0,)), dirty=True)


def _vce(a, b):
    if b.ispad:
        return a, b
    if a.ispad:
        return b, a
    return (_Vr(jnp.minimum(a.k, b.k)), _Vr(jnp.maximum(a.k, b.k)))


def _bitonic_merge(xs):
    n = len(xs)
    a = n // 2
    xs = xs[:a] + [_vrev(xs[n - 1 - i]) for i in range(a)]
    d = a
    while d >= 1:
        for i in range(n):
            if (i // d) % 2 == 0:
                xs[i], xs[i + d] = _vce(xs[i], xs[i + d])
        d //= 2
    return [_vsort(x) for x in xs]


def _sort_row(xs):
    nv = len(xs)
    xs = [_vsort(x) for x in xs]
    run = 1
    while run < nv:
        nxs = []
        for s in range(0, nv, 2 * run):
            nxs += _bitonic_merge(xs[s:s + 2 * run])
        xs = nxs
        run *= 2
    return xs


def _make_sc_call(B, L, chunk):
    rows_per_w = B // _NW
    n_chunks = rows_per_w // chunk
    nfull = L // _LANES
    tail = L - nfull * _LANES
    nreal = nfull + (1 if tail else 0)
    ebuf_sz = 256
    mesh = plsc.VectorSubcoreMesh(
        core_axis_name="c", subcore_axis_name="s",
        num_cores=_NC, num_subcores=_NS)

    @functools.partial(
        pl.kernel,
        out_type=jax.ShapeDtypeStruct((_NW * _LANES,), jnp.float32),
        mesh=mesh,
        compiler_params=pltpu.CompilerParams(
            needs_layout_passes=False, use_tc_tiling_on_sc=True),
        scratch_types=[
            pltpu.VMEM((chunk, L), jnp.float32),
            pltpu.VMEM((chunk, L), jnp.float32),
            pltpu.VMEM((ebuf_sz,), jnp.float32),
            pltpu.VMEM((_TLEN,), jnp.float32),
            pltpu.VMEM((_TLEN,), jnp.float32),
            pltpu.VMEM((_LANES,), jnp.float32),
        ],
    )
    def sc_kernel(yp_hbm, yt_hbm, logt_hbm, logd_hbm, out_hbm,
                  pbuf, tbuf, ebuf, logt, logd, outv):
        wid = lax.axis_index("s") * _NC + lax.axis_index("c")
        base = wid * rows_per_w
        lane = lax.iota(jnp.int32, 16)
        dup = lane < (_LANES - tail)
        pltpu.sync_copy(logt_hbm, logt)
        pltpu.sync_copy(logd_hbm, logd)
        for off in range(nfull * _LANES, ebuf_sz, _LANES):
            ebuf[pl.ds(off, _LANES)] = jnp.zeros((_LANES,), jnp.float32)

        def chunk_body(ci, acc):
            row0 = base + ci * chunk
            pltpu.sync_copy(yt_hbm.at[pl.ds(row0, chunk)], tbuf)
            pltpu.sync_copy(yp_hbm.at[pl.ds(row0, chunk)], pbuf)

            def row_body(r, acc2):
                ts, pms = [], []
                for i in range(nreal):
                    off = i * _LANES if i < nfull else L - _LANES
                    t = tbuf[r, pl.ds(off, _LANES)]
                    p = pbuf[r, pl.ds(off, _LANES)]
                    pm = jnp.where(t == _PAD, _NEG_INF, p)
                    if i == nfull:
                        pm = jnp.where(dup, _NEG_INF, pm)
                    ts.append(t)
                    pms.append(pm)
                mv = pms[0]
                for i in range(1, nreal):
                    mv = jnp.maximum(mv, pms[i])
                m = jnp.max(mv)
                lin = None
                evs = []
                for i in range(nreal):
                    gone = pms[i] == _NEG_INF
                    evs.append(jnp.where(gone, np.float32(0.0),
                                         jnp.exp(pms[i] - m)))
                    term = jnp.where(gone, np.float32(0.0), pms[i] - m)
                    lin = term if lin is None else lin + term
                order = ([nfull] if tail else []) + list(range(nfull))
                for i in order:
                    off = i * _LANES if i < nfull else L - _LANES
                    ebuf[pl.ds(off, _LANES)] = evs[i]
                xs = []
                for i in range(nreal):
                    off = i * _LANES if i < nfull else L - _LANES
                    kb = (plsc.bitcast(ts[i], jnp.int32) & _HI24) \
                        | (lane + np.int32(off))
                    key = plsc.bitcast(kb, jnp.float32)
                    if i == nfull:
                        key = jnp.where(dup, _PAD_KEY, key)
                    xs.append(_Vr(key))
                pad_k = jnp.full((_LANES,), _PAD_KEY, jnp.float32)
                xs += [_Vr(pad_k, ispad=True, dirty=False)
                       for _ in range(_NV - nreal)]
                xs = _sort_row(xs)
                carry = np.float32(0.0)
                logsum = jnp.zeros((_LANES,), jnp.float32)
                nlast = nreal - 1
                for pos, x in enumerate(xs):
                    if x.ispad:
                        continue
                    kb = plsc.bitcast(x.k, jnp.int32)
                    es = plsc.load_gather(ebuf, [kb & _LO8])
                    h = kb & _HI24
                    if pos == nlast:
                        isinf = h == _INFBITS
                        es = jnp.where(isinf, np.float32(0.0), es)
                        valid = (h != _PADBITS) & (~isinf)
                    else:
                        valid = h != _PADBITS
                    cs = plsc.cumsum(es) + carry
                    carry = carry + jnp.sum(es)
                    lbits = plsc.bitcast(cs + _EPS, jnp.int32)
                    ti = (lbits >> 15) - np.int32(_TBASE)
                    fl = (lbits & np.int32(0x7FFF)).astype(jnp.float32)
                    lg = (plsc.load_gather(logt, [ti])
                          + fl * plsc.load_gather(logd, [ti]))
                    logsum = logsum + jnp.where(valid, lg, np.float32(0.0))
                return acc2 + (logsum - lin)

            return lax.fori_loop(0, chunk, row_body, acc, unroll=2)

        acc = lax.fori_loop(0, n_chunks, chunk_body,
                            jnp.zeros((_LANES,), jnp.float32))
        outv[...] = acc
        pltpu.sync_copy(outv, out_hbm.at[pl.ds(wid * _LANES, _LANES)])

    return sc_kernel


def kernel(y_pred, y_true):
    B, L = y_pred.shape
    sc_call = _make_sc_call(B, L, chunk=64)
    logt, logd = _log_tables()
    partials = sc_call(y_pred, y_true, jnp.asarray(logt), jnp.asarray(logd))
    return jnp.sum(partials) / np.float32(B)


# speedup vs baseline: 1.0926x; 1.0926x over previous
import functools

import numpy as np
import jax
import jax.numpy as jnp
from jax import lax
from jax.experimental import pallas as pl
from jax.experimental.pallas import tpu as pltpu
from jax.experimental.pallas import tpu_sc as plsc

_EPS = np.float32(1e-06)
_PAD = np.float32(-1.0)
_NEG_INF = np.float32(-np.inf)
_LN2 = np.float32(0.6931471805599453)
_C3 = np.float32(1.0 / 3.0)
_C5 = np.float32(1.0 / 5.0)
_C7 = np.float32(1.0 / 7.0)
_C9 = np.float32(1.0 / 9.0)

_NC, _NS, _LANES = 2, 16, 16
_NW = _NC * _NS
_NV = 16

_PAD_KEY = np.float32(np.inf)
_PADBITS = np.int32(np.int64(0xBF800000) - (1 << 32))
_INFBITS = np.int32(0x7F800000)
_HI24 = np.int32(np.int64(0xFFFFFF00) - (1 << 32))
_LO8 = np.int32(0xFF)


def _swlog(x):
    bits = plsc.bitcast(x, jnp.int32)
    e = ((bits >> 23) & 0xFF) - 127
    mant = plsc.bitcast((bits & 0x007FFFFF) | 0x3F800000, jnp.float32)
    z = (mant - np.float32(1.0)) / (mant + np.float32(1.0))
    w = z * z
    poly = (np.float32(2.0) * z *
            (np.float32(1.0) + w * (_C3 + w * (_C5 + w * (_C7 + w * _C9)))))
    return e.astype(jnp.float32) * _LN2 + poly


_TBASE = int(np.float32(1e-6).view(np.int32)) >> 15
_TLEN = 7168


def _log_tables():
    idx = np.arange(_TLEN, dtype=np.int64)
    bits0 = (idx + _TBASE) << 15
    x0 = bits0.astype(np.int64).astype(np.int32).view(np.float32).astype(np.float64)
    x1 = (bits0 + 32768).astype(np.int64).astype(np.int32).view(np.float32).astype(np.float64)
    t = np.log(x0)
    d = (np.log(x1) - t) / 32768.0
    return t.astype(np.float32), d.astype(np.float32)


class _Vr:
    __slots__ = ("k", "ispad", "dirty")

    def __init__(self, k, ispad=False, dirty=True):
        self.k, self.ispad, self.dirty = k, ispad, dirty


def _vsort(x):
    if x.ispad or not x.dirty:
        return x
    return _Vr(jnp.sort(x.k), dirty=False)


def _vrev(x):
    if x.ispad:
        return x
    return _Vr(lax.rev(x.k, (0,)), dirty=True)


def _vce(a, b):
    if b.ispad:
        return a, b
    if a.ispad:
        return b, a
    return (_Vr(jnp.minimum(a.k, b.k)), _Vr(jnp.maximum(a.k, b.k)))


def _bitonic_merge(xs):
    n = len(xs)
    a = n // 2
    xs = xs[:a] + [_vrev(xs[n - 1 - i]) for i in range(a)]
    d = a
    while d >= 1:
        for i in range(n):
            if (i // d) % 2 == 0:
                xs[i], xs[i + d] = _vce(xs[i], xs[i + d])
        d //= 2
    return [_vsort(x) for x in xs]


def _sort_row(xs):
    nv = len(xs)
    xs = [_vsort(x) for x in xs]
    run = 1
    while run < nv:
        nxs = []
        for s in range(0, nv, 2 * run):
            nxs += _bitonic_merge(xs[s:s + 2 * run])
        xs = nxs
        run *= 2
    return xs


def _make_sc_call(B, L, chunk):
    rows_per_w = B // _NW
    n_chunks = rows_per_w // chunk
    nfull = L // _LANES
    tail = L - nfull * _LANES
    nreal = nfull + (1 if tail else 0)
    ebuf_sz = 256
    mesh = plsc.VectorSubcoreMesh(
        core_axis_name="c", subcore_axis_name="s",
        num_cores=_NC, num_subcores=_NS)

    @functools.partial(
        pl.kernel,
        out_type=jax.ShapeDtypeStruct((_NW * _LANES,), jnp.float32),
        mesh=mesh,
        compiler_params=pltpu.CompilerParams(
            needs_layout_passes=False, use_tc_tiling_on_sc=True),
        scratch_types=[
            pltpu.VMEM((chunk, L), jnp.float32),
            pltpu.VMEM((chunk, L), jnp.float32),
            pltpu.VMEM((ebuf_sz,), jnp.float32),
            pltpu.VMEM((_TLEN,), jnp.float32),
            pltpu.VMEM((_TLEN,), jnp.float32),
            pltpu.VMEM((_LANES,), jnp.float32),
        ],
    )
    def sc_kernel(yp_hbm, yt_hbm, logt_hbm, logd_hbm, out_hbm,
                  pbuf, tbuf, ebuf, logt, logd, outv):
        wid = lax.axis_index("s") * _NC + lax.axis_index("c")
        base = wid * rows_per_w
        lane = lax.iota(jnp.int32, 16)
        dup = lane < (_LANES - tail)
        pltpu.sync_copy(logt_hbm, logt)
        pltpu.sync_copy(logd_hbm, logd)
        for off in range(nfull * _LANES, ebuf_sz, _LANES):
            ebuf[pl.ds(off, _LANES)] = jnp.zeros((_LANES,), jnp.float32)

        def chunk_body(ci, acc):
            row0 = base + ci * chunk
            pltpu.sync_copy(yt_hbm.at[pl.ds(row0, chunk)], tbuf)
            pltpu.sync_copy(yp_hbm.at[pl.ds(row0, chunk)], pbuf)

            def row_body(r, acc2):
                ts, pms = [], []
                for i in range(nreal):
                    off = i * _LANES if i < nfull else L - _LANES
                    t = tbuf[r, pl.ds(off, _LANES)]
                    p = pbuf[r, pl.ds(off, _LANES)]
                    pm = jnp.where(t == _PAD, _NEG_INF, p)
                    if i == nfull:
                        pm = jnp.where(dup, _NEG_INF, pm)
                    ts.append(t)
                    pms.append(pm)
                mv = pms[0]
                for i in range(1, nreal):
                    mv = jnp.maximum(mv, pms[i])
                m = jnp.max(mv)
                lin = None
                evs = []
                for i in range(nreal):
                    gone = pms[i] == _NEG_INF
                    evs.append(jnp.where(gone, np.float32(0.0),
                                         jnp.exp(pms[i] - m)))
                    term = jnp.where(gone, np.float32(0.0), pms[i] - m)
                    lin = term if lin is None else lin + term
                order = ([nfull] if tail else []) + list(range(nfull))
                for i in order:
                    off = i * _LANES if i < nfull else L - _LANES
                    ebuf[pl.ds(off, _LANES)] = evs[i]
                xs = []
                for i in range(nreal):
                    off = i * _LANES if i < nfull else L - _LANES
                    kb = (plsc.bitcast(ts[i], jnp.int32) & _HI24) \
                        | (lane + np.int32(off))
                    key = plsc.bitcast(kb, jnp.float32)
                    if i == nfull:
                        key = jnp.where(dup, _PAD_KEY, key)
                    xs.append(_Vr(key))
                pad_k = jnp.full((_LANES,), _PAD_KEY, jnp.float32)
                xs += [_Vr(pad_k, ispad=True, dirty=False)
                       for _ in range(_NV - nreal)]
                xs = _sort_row(xs)
                carry = jnp.full((_LANES,), _EPS, jnp.float32)
                lane15 = jnp.full((_LANES,), 15, jnp.int32)
                logsum = jnp.zeros((_LANES,), jnp.float32)
                nlast = nreal - 1
                for pos, x in enumerate(xs):
                    if x.ispad:
                        continue
                    kb = plsc.bitcast(x.k, jnp.int32)
                    es = plsc.load_gather(ebuf, [kb & _LO8])
                    h = kb & _HI24
                    if pos == nlast:
                        isinf = h == _INFBITS
                        es = jnp.where(isinf, np.float32(0.0), es)
                        valid = (h != _PADBITS) & (~isinf)
                    else:
                        valid = h != _PADBITS
                    cs = plsc.cumsum(es) + carry
                    if pos != nlast:
                        carry = cs.at[lane15].get(mode="promise_in_bounds")
                    lbits = plsc.bitcast(cs, jnp.int32)
                    ti = (lbits >> 15) - np.int32(_TBASE)
                    fl = (lbits & np.int32(0x7FFF)).astype(jnp.float32)
                    lg = (plsc.load_gather(logt, [ti])
                          + fl * plsc.load_gather(logd, [ti]))
                    logsum = logsum + jnp.where(valid, lg, np.float32(0.0))
                return acc2 + (logsum - lin)

            return lax.fori_loop(0, chunk, row_body, acc, unroll=2)

        acc = lax.fori_loop(0, n_chunks, chunk_body,
                            jnp.zeros((_LANES,), jnp.float32))
        outv[...] = acc
        pltpu.sync_copy(outv, out_hbm.at[pl.ds(wid * _LANES, _LANES)])

    return sc_kernel


def kernel(y_pred, y_true):
    B, L = y_pred.shape
    sc_call = _make_sc_call(B, L, chunk=64)
    logt, logd = _log_tables()
    partials = sc_call(y_pred, y_true, jnp.asarray(logt), jnp.asarray(logd))
    return jnp.sum(partials) / np.float32(B)
